# baseline (speedup 1.0000x reference)
import functools

import jax
import jax.numpy as jnp
from jax import lax
from jax.experimental import pallas as pl
from jax.experimental.pallas import tpu as pltpu
from jax.experimental.pallas import tpu_sc as plsc

L = 10
DIM = 4
LOG2T = 19
TSIZE = 2 ** LOG2T
BASE_RES = 16.0
SCALE_MULTI = 0.5
HIDDEN = 64

NC = 2
NS = 16
NW = NC * NS

TR = 3968
TPAD = TR * 128


def _sc_gather_planes(tab1d, idx2d, chunk):
    n = idx2d.shape[0] * 128
    per_w = n // NW
    assert per_w % chunk == 0 and chunk % 128 == 0
    nchunks = per_w // chunk
    nrow = chunk // 128
    mesh = plsc.VectorSubcoreMesh(core_axis_name="c", subcore_axis_name="s")

    @functools.partial(
        pl.kernel,
        out_type=jax.ShapeDtypeStruct((4 * n,), jnp.float32),
        mesh=mesh,
        scratch_types=[
            pltpu.VMEM((nrow, 128), jnp.int32),
            pltpu.VMEM((4 * chunk,), jnp.int32),
            pltpu.VMEM((4 * chunk,), jnp.float32),
            pltpu.VMEM((4, chunk), jnp.float32),
            pltpu.SemaphoreType.DMA,
        ],
        compiler_params=pltpu.CompilerParams(use_tc_tiling_on_sc=False,
                                             needs_layout_passes=False),
    )
    def k(tab_hbm, idx_hbm, out_hbm, idx_v, gidx, rows_v, planes_v, sem):
        wid = lax.axis_index("s") * NC + lax.axis_index("c")
        base = wid * per_w
        lane4 = lax.iota(jnp.int32, 16) * 4

        def body(j, carry):
            off = base + j * chunk
            pltpu.sync_copy(idx_hbm.at[pl.ds(off // 128, nrow)], idx_v)

            def expand(g, c2):
                for v in range(8):
                    h4 = idx_v[g, pl.ds(v * 16, 16)]
                    pos = lane4 + (g * 512 + v * 64)
                    for d in range(4):
                        plsc.store_scatter(gidx, [pos + d], h4 + d)
                return c2

            lax.fori_loop(0, nrow, expand, 0)
            pltpu.async_copy(tab_hbm.at[gidx], rows_v, sem).wait()

            def deint(g, c2):
                for v in range(8):
                    pos = lane4 + (g * 512 + v * 64)
                    for d in range(4):
                        vals = plsc.load_gather(rows_v, [pos + d])
                        planes_v[d, pl.ds(g * 128 + v * 16, 16)] = vals
                return c2

            lax.fori_loop(0, nrow, deint, 0)
            for d in range(4):
                pltpu.sync_copy(planes_v.at[d],
                                out_hbm.at[pl.ds(d * n + off, chunk)])
            return carry

        lax.fori_loop(0, nchunks, body, 0)

    return k(tab1d, idx2d)


def _erf(x):
    t = 1.0 / (1.0 + 0.3275911 * x)
    poly = t * (0.254829592 + t * (-0.284496736 + t * (1.421413741
                + t * (-1.453152027 + t * 1.061405429))))
    return 1.0 - poly * jnp.exp(-x * x)


def _geom_kernel(xyz_ref, cs_ref, idx_ref, w_ref):
    inv_s = 1.0 / cs_ref[3]
    cen = (cs_ref[0], cs_ref[1], cs_ref[2])
    a = [(xyz_ref[0, i] - cen[i]) * inv_s for i in range(3)]
    d1 = [(xyz_ref[1, i] - cen[i]) * inv_s - a[i] for i in range(3)]
    d2 = [(xyz_ref[2, i] - cen[i]) * inv_s - a[i] for i in range(3)]
    d3 = [(xyz_ref[3, i] - cen[i]) * inv_s - a[i] for i in range(3)]
    eps = jnp.float32(1e-8)
    a11 = 2.0 * d1[0] + eps
    a12 = 2.0 * d1[1]
    a13 = 2.0 * d1[2]
    a21 = 2.0 * d2[0]
    a22 = 2.0 * d2[1] + eps
    a23 = 2.0 * d2[2]
    a31 = 2.0 * d3[0]
    a32 = 2.0 * d3[1]
    a33 = 2.0 * d3[2] + eps
    r1 = d1[0] * d1[0] + d1[1] * d1[1] + d1[2] * d1[2]
    r2 = d2[0] * d2[0] + d2[1] * d2[1] + d2[2] * d2[2]
    r3 = d3[0] * d3[0] + d3[1] * d3[1] + d3[2] * d3[2]
    c11 = a22 * a33 - a23 * a32
    c12 = a23 * a31 - a21 * a33
    c13 = a21 * a32 - a22 * a31
    det = a11 * c11 + a12 * c12 + a13 * c13
    inv = 1.0 / det
    ox = (r1 * c11 + r2 * (a13 * a32 - a12 * a33) + r3 * (a12 * a23 - a13 * a22)) * inv
    oy = (r1 * c12 + r2 * (a11 * a33 - a13 * a31) + r3 * (a13 * a21 - a11 * a23)) * inv
    oz = (r1 * c13 + r2 * (a12 * a31 - a11 * a32) + r3 * (a11 * a22 - a12 * a21)) * inv
    cx = a[0] + ox
    cy = a[1] + oy
    cz = a[2] + oz
    radius = jnp.sqrt(ox * ox + oy * oy + oz * oz)
    nrm = jnp.sqrt(cx * cx + cy * cy + cz * cz)
    safe = jnp.maximum(nrm, 1.0)
    fac = jnp.where(nrm <= 1.0, 1.0, (2.0 - 1.0 / safe) / safe)
    cr = radius / (safe * safe) * SCALE_MULTI
    xs0 = jnp.clip((cx * fac * 0.5 + 1.0) * 0.5, 0.0, 1.0 - 1e-6)
    xs1 = jnp.clip((cy * fac * 0.5 + 1.0) * 0.5, 0.0, 1.0 - 1e-6)
    xs2 = jnp.clip((cz * fac * 0.5 + 1.0) * 0.5, 0.0, 1.0 - 1e-6)
    p1 = jnp.int32(-1640531535)
    p2 = jnp.int32(805459861)
    mask = jnp.int32(TSIZE - 1)
    for l in range(L):
        res = BASE_RES * (2.0 ** l)
        gx = xs0 * res
        gy = xs1 * res
        gz = xs2 * res
        fx = jnp.floor(gx)
        fy = jnp.floor(gy)
        fz = jnp.floor(gz)
        wx1 = gx - fx
        wy1 = gy - fy
        wz1 = gz - fz
        wx0 = 1.0 - wx1
        wy0 = 1.0 - wy1
        wz0 = 1.0 - wz1
        ix = fx.astype(jnp.int32)
        iy = fy.astype(jnp.int32)
        iz = fz.astype(jnp.int32)
        hx = (ix, ix + 1)
        hy0 = iy * p1
        hy = (hy0, hy0 + p1)
        hz0 = iz * p2
        hz = (hz0, hz0 + p2)
        wxs = (wx0, wx1)
        wys = (wy0, wy1)
        wzs = (wz0, wz1)
        m = jnp.maximum(jnp.float32(8.0 * l) * cr, 1e-12)
        scal = _erf(jax.lax.rsqrt(m))
        base = jnp.int32(l * TSIZE)
        for c in range(8):
            bx = (c >> 0) & 1
            by = (c >> 1) & 1
            bz = (c >> 2) & 1
            h = ((hx[bx] ^ hy[by] ^ hz[bz]) & mask) + base
            wc = wxs[bx] * wys[by] * wzs[bz] * scal
            idx_ref[c * L + l] = h * 4
            w_ref[c * L + l] = wc


def _mlp_kernel(raw_ref, w_ref,
                W1t_ref, b1_ref, W2t_ref, b2_ref, W3t_ref, b3_ref, out_ref):
    s = w_ref.shape[1]
    j_i = lax.broadcasted_iota(jnp.int32, (4 * L, 8 * L), 0)
    cl_i = lax.broadcasted_iota(jnp.int32, (4 * L, 8 * L), 1)
    w = w_ref[:, :]
    featsT = jnp.zeros((4 * L, s), jnp.float32)
    for d in range(4):
        Gt = ((j_i % 4 == d) & (j_i // 4 == cl_i % L)).astype(jnp.float32)
        featsT = featsT + jnp.dot(Gt, w * raw_ref[d],
                                  preferred_element_type=jnp.float32)
    scale = jnp.float32(1.0507009873554805)
    alpha = jnp.float32(1.6732632423543772)

    def selu(x):
        return scale * jnp.where(x > 0, x, alpha * (jnp.exp(x) - 1.0))

    h = selu(jnp.dot(W1t_ref[:, :], featsT, preferred_element_type=jnp.float32)
             + b1_ref[:, :])
    h = selu(jnp.dot(W2t_ref[:, :], h, preferred_element_type=jnp.float32)
             + b2_ref[:, :])
    out_ref[:, :] = (jnp.dot(W3t_ref[:, :], h, preferred_element_type=jnp.float32)
                     + b3_ref[:, :])


def kernel(vertices, indices, tables, W1, b1, W2, b2, W3, b3, center, scene_scaling):
    T = indices.shape[0]
    idx32 = indices.astype(jnp.int32) * 4
    idx_pad = jnp.pad(idx32, ((0, TPAD - T), (0, 0)))
    verts1d = jnp.pad(vertices, ((0, 0), (0, 1))).reshape(-1)

    nv = 4 * TPAD
    vflat = idx_pad.reshape(nv // 128, 128)
    vgp = _sc_gather_planes(verts1d, vflat, chunk=2048)
    xyz = vgp.reshape(4, TPAD, 4).transpose(2, 0, 1)[:, :3, :]
    xyz = xyz.reshape(4, 3, TR, 128)
    cs = jnp.concatenate([center.reshape(3), scene_scaling.reshape(1)])

    S = 64
    grid = TR // S
    idx80, w80 = pl.pallas_call(
        _geom_kernel,
        grid=(grid,),
        in_specs=[
            pl.BlockSpec((4, 3, S, 128), lambda t: (0, 0, t, 0)),
            pl.BlockSpec(memory_space=pltpu.SMEM),
        ],
        out_specs=[
            pl.BlockSpec((8 * L, S, 128), lambda t: (0, t, 0)),
            pl.BlockSpec((8 * L, S, 128), lambda t: (0, t, 0)),
        ],
        out_shape=[
            jax.ShapeDtypeStruct((8 * L, TR, 128), jnp.int32),
            jax.ShapeDtypeStruct((8 * L, TR, 128), jnp.float32),
        ],
    )(xyz, cs)

    nt = TPAD * 8 * L
    idx_t = idx80.reshape(nt // 128, 128)
    w_t = w80.reshape(8 * L, TPAD)

    tab1d = tables.reshape(-1)
    rawp = _sc_gather_planes(tab1d, idx_t, chunk=8192)
    raw = rawp.reshape(4, 8 * L, TPAD)

    SD = 512
    W1t = W1.transpose(1, 0)
    W2t = W2.transpose(1, 0)
    W3t = jnp.pad(W3, ((0, 0), (0, 7))).transpose(1, 0)
    b3p = jnp.pad(b3, (0, 7))
    out = pl.pallas_call(
        _mlp_kernel,
        grid=(TPAD // SD,),
        in_specs=[
            pl.BlockSpec((4, 8 * L, SD), lambda t: (0, 0, t)),
            pl.BlockSpec((8 * L, SD), lambda t: (0, t)),
            pl.BlockSpec((HIDDEN, L * DIM), lambda t: (0, 0)),
            pl.BlockSpec((HIDDEN, 1), lambda t: (0, 0)),
            pl.BlockSpec((HIDDEN, HIDDEN), lambda t: (0, 0)),
            pl.BlockSpec((HIDDEN, 1), lambda t: (0, 0)),
            pl.BlockSpec((8, HIDDEN), lambda t: (0, 0)),
            pl.BlockSpec((8, 1), lambda t: (0, 0)),
        ],
        out_specs=pl.BlockSpec((8, SD), lambda t: (0, t)),
        out_shape=jax.ShapeDtypeStruct((8, TPAD), jnp.float32),
    )(raw, w_t, W1t, b1.reshape(HIDDEN, 1), W2t, b2.reshape(HIDDEN, 1),
      W3t, b3p.reshape(8, 1))

    return out[0, :T].reshape(T, 1)


# speedup vs baseline: 1.7714x; 1.7714x over previous
import functools

import jax
import jax.numpy as jnp
from jax import lax
from jax.experimental import pallas as pl
from jax.experimental.pallas import tpu as pltpu
from jax.experimental.pallas import tpu_sc as plsc

L = 10
DIM = 4
LOG2T = 19
TSIZE = 2 ** LOG2T
BASE_RES = 16.0
SCALE_MULTI = 0.5
HIDDEN = 64

NC = 2
NS = 16
NW = NC * NS

TR = 3968
TPAD = TR * 128


def _sc_gather_planes(tab1d, idx2d, chunk, nplanes, stride):
    n = idx2d.shape[0] * 128
    per_w = n // NW
    assert per_w % chunk == 0 and chunk % 128 == 0
    nchunks = per_w // chunk
    nrow = chunk // 128
    mesh = plsc.VectorSubcoreMesh(core_axis_name="c", subcore_axis_name="s")

    @functools.partial(
        pl.kernel,
        out_type=jax.ShapeDtypeStruct((nplanes * n,), jnp.float32),
        mesh=mesh,
        scratch_types=[
            pltpu.VMEM((nrow, 128), jnp.int32),
            pltpu.VMEM((nplanes, chunk), jnp.int32),
            pltpu.VMEM((nplanes, chunk), jnp.float32),
            pltpu.SemaphoreType.DMA,
        ],
        compiler_params=pltpu.CompilerParams(use_tc_tiling_on_sc=False,
                                             needs_layout_passes=False),
    )
    def k(tab_hbm, idx_hbm, out_hbm, idx_v, gidx, planes_v, sem):
        wid = lax.axis_index("s") * NC + lax.axis_index("c")
        base = wid * per_w

        def body(j, carry):
            off = base + j * chunk
            pltpu.sync_copy(idx_hbm.at[pl.ds(off // 128, nrow)], idx_v)

            def expand(g, c2):
                for v in range(8):
                    h = idx_v[g, pl.ds(v * 16, 16)]
                    for d in range(nplanes):
                        gidx[d, pl.ds(g * 128 + v * 16, 16)] = h + d * stride
                return c2

            lax.fori_loop(0, nrow, expand, 0)
            copies = [
                pltpu.async_copy(tab_hbm.at[gidx.at[d]], planes_v.at[d], sem)
                for d in range(nplanes)
            ]
            for cp in copies:
                cp.wait()
            for d in range(nplanes):
                pltpu.sync_copy(planes_v.at[d],
                                out_hbm.at[pl.ds(d * n + off, chunk)])
            return carry

        lax.fori_loop(0, nchunks, body, 0)

    return k(tab1d, idx2d)


def _erf(x):
    t = 1.0 / (1.0 + 0.3275911 * x)
    poly = t * (0.254829592 + t * (-0.284496736 + t * (1.421413741
                + t * (-1.453152027 + t * 1.061405429))))
    return 1.0 - poly * jnp.exp(-x * x)


def _geom_kernel(xyz_ref, cs_ref, idx_ref, w_ref):
    inv_s = 1.0 / cs_ref[3]
    cen = (cs_ref[0], cs_ref[1], cs_ref[2])
    a = [(xyz_ref[0, i] - cen[i]) * inv_s for i in range(3)]
    d1 = [(xyz_ref[1, i] - cen[i]) * inv_s - a[i] for i in range(3)]
    d2 = [(xyz_ref[2, i] - cen[i]) * inv_s - a[i] for i in range(3)]
    d3 = [(xyz_ref[3, i] - cen[i]) * inv_s - a[i] for i in range(3)]
    eps = jnp.float32(1e-8)
    a11 = 2.0 * d1[0] + eps
    a12 = 2.0 * d1[1]
    a13 = 2.0 * d1[2]
    a21 = 2.0 * d2[0]
    a22 = 2.0 * d2[1] + eps
    a23 = 2.0 * d2[2]
    a31 = 2.0 * d3[0]
    a32 = 2.0 * d3[1]
    a33 = 2.0 * d3[2] + eps
    r1 = d1[0] * d1[0] + d1[1] * d1[1] + d1[2] * d1[2]
    r2 = d2[0] * d2[0] + d2[1] * d2[1] + d2[2] * d2[2]
    r3 = d3[0] * d3[0] + d3[1] * d3[1] + d3[2] * d3[2]
    c11 = a22 * a33 - a23 * a32
    c12 = a23 * a31 - a21 * a33
    c13 = a21 * a32 - a22 * a31
    det = a11 * c11 + a12 * c12 + a13 * c13
    inv = 1.0 / det
    ox = (r1 * c11 + r2 * (a13 * a32 - a12 * a33) + r3 * (a12 * a23 - a13 * a22)) * inv
    oy = (r1 * c12 + r2 * (a11 * a33 - a13 * a31) + r3 * (a13 * a21 - a11 * a23)) * inv
    oz = (r1 * c13 + r2 * (a12 * a31 - a11 * a32) + r3 * (a11 * a22 - a12 * a21)) * inv
    cx = a[0] + ox
    cy = a[1] + oy
    cz = a[2] + oz
    radius = jnp.sqrt(ox * ox + oy * oy + oz * oz)
    nrm = jnp.sqrt(cx * cx + cy * cy + cz * cz)
    safe = jnp.maximum(nrm, 1.0)
    fac = jnp.where(nrm <= 1.0, 1.0, (2.0 - 1.0 / safe) / safe)
    cr = radius / (safe * safe) * SCALE_MULTI
    xs0 = jnp.clip((cx * fac * 0.5 + 1.0) * 0.5, 0.0, 1.0 - 1e-6)
    xs1 = jnp.clip((cy * fac * 0.5 + 1.0) * 0.5, 0.0, 1.0 - 1e-6)
    xs2 = jnp.clip((cz * fac * 0.5 + 1.0) * 0.5, 0.0, 1.0 - 1e-6)
    p1 = jnp.int32(-1640531535)
    p2 = jnp.int32(805459861)
    mask = jnp.int32(TSIZE - 1)
    for l in range(L):
        res = BASE_RES * (2.0 ** l)
        gx = xs0 * res
        gy = xs1 * res
        gz = xs2 * res
        fx = jnp.floor(gx)
        fy = jnp.floor(gy)
        fz = jnp.floor(gz)
        wx1 = gx - fx
        wy1 = gy - fy
        wz1 = gz - fz
        wx0 = 1.0 - wx1
        wy0 = 1.0 - wy1
        wz0 = 1.0 - wz1
        ix = fx.astype(jnp.int32)
        iy = fy.astype(jnp.int32)
        iz = fz.astype(jnp.int32)
        hx = (ix, ix + 1)
        hy0 = iy * p1
        hy = (hy0, hy0 + p1)
        hz0 = iz * p2
        hz = (hz0, hz0 + p2)
        wxs = (wx0, wx1)
        wys = (wy0, wy1)
        wzs = (wz0, wz1)
        m = jnp.maximum(jnp.float32(8.0 * l) * cr, 1e-12)
        scal = _erf(jax.lax.rsqrt(m))
        base = jnp.int32(l * TSIZE)
        for c in range(8):
            bx = (c >> 0) & 1
            by = (c >> 1) & 1
            bz = (c >> 2) & 1
            h = ((hx[bx] ^ hy[by] ^ hz[bz]) & mask) + base
            wc = wxs[bx] * wys[by] * wzs[bz] * scal
            idx_ref[c * L + l] = h
            w_ref[c * L + l] = wc


def _mlp_kernel(raw_ref, w_ref,
                W1t_ref, b1_ref, W2t_ref, b2_ref, W3t_ref, b3_ref, out_ref):
    s = w_ref.shape[1]
    j_i = lax.broadcasted_iota(jnp.int32, (4 * L, 8 * L), 0)
    cl_i = lax.broadcasted_iota(jnp.int32, (4 * L, 8 * L), 1)
    w = w_ref[:, :]
    featsT = jnp.zeros((4 * L, s), jnp.float32)
    for d in range(4):
        Gt = ((j_i % 4 == d) & (j_i // 4 == cl_i % L)).astype(jnp.float32)
        featsT = featsT + jnp.dot(Gt, w * raw_ref[d],
                                  preferred_element_type=jnp.float32)
    scale = jnp.float32(1.0507009873554805)
    alpha = jnp.float32(1.6732632423543772)

    def selu(x):
        return scale * jnp.where(x > 0, x, alpha * (jnp.exp(x) - 1.0))

    h = selu(jnp.dot(W1t_ref[:, :], featsT, preferred_element_type=jnp.float32)
             + b1_ref[:, :])
    h = selu(jnp.dot(W2t_ref[:, :], h, preferred_element_type=jnp.float32)
             + b2_ref[:, :])
    out_ref[:, :] = (jnp.dot(W3t_ref[:, :], h, preferred_element_type=jnp.float32)
                     + b3_ref[:, :])


def kernel(vertices, indices, tables, W1, b1, W2, b2, W3, b3, center, scene_scaling):
    T = indices.shape[0]
    idx32 = indices.astype(jnp.int32)
    idx_pad = jnp.pad(idx32, ((0, TPAD - T), (0, 0)))
    VP = 200064
    verts1d = jnp.pad(vertices, ((0, VP - vertices.shape[0]), (0, 0)))
    verts1d = verts1d.transpose(1, 0).reshape(3, VP // 128, 128).reshape(-1)

    nv = 4 * TPAD
    vflat = idx_pad.reshape(nv // 128, 128)
    vgp = _sc_gather_planes(verts1d, vflat, chunk=2048, nplanes=3, stride=VP)
    xyz = vgp.reshape(3, TPAD, 4).transpose(2, 0, 1)
    xyz = xyz.reshape(4, 3, TR, 128)
    cs = jnp.concatenate([center.reshape(3), scene_scaling.reshape(1)])

    S = 64
    grid = TR // S
    idx80, w80 = pl.pallas_call(
        _geom_kernel,
        grid=(grid,),
        in_specs=[
            pl.BlockSpec((4, 3, S, 128), lambda t: (0, 0, t, 0)),
            pl.BlockSpec(memory_space=pltpu.SMEM),
        ],
        out_specs=[
            pl.BlockSpec((8 * L, S, 128), lambda t: (0, t, 0)),
            pl.BlockSpec((8 * L, S, 128), lambda t: (0, t, 0)),
        ],
        out_shape=[
            jax.ShapeDtypeStruct((8 * L, TR, 128), jnp.int32),
            jax.ShapeDtypeStruct((8 * L, TR, 128), jnp.float32),
        ],
    )(xyz, cs)

    nt = TPAD * 8 * L
    idx_t = idx80.reshape(nt // 128, 128)
    w_t = w80.reshape(8 * L, TPAD)

    LT = L * TSIZE
    tab1d = tables.reshape(LT, 4).transpose(1, 0)
    tab1d = tab1d.reshape(4, LT // 128, 128).reshape(-1)
    rawp = _sc_gather_planes(tab1d, idx_t, chunk=8192, nplanes=4, stride=LT)
    raw = rawp.reshape(4, 8 * L, TPAD)

    SD = 512
    W1t = W1.transpose(1, 0)
    W2t = W2.transpose(1, 0)
    W3t = jnp.pad(W3, ((0, 0), (0, 7))).transpose(1, 0)
    b3p = jnp.pad(b3, (0, 7))
    out = pl.pallas_call(
        _mlp_kernel,
        grid=(TPAD // SD,),
        in_specs=[
            pl.BlockSpec((4, 8 * L, SD), lambda t: (0, 0, t)),
            pl.BlockSpec((8 * L, SD), lambda t: (0, t)),
            pl.BlockSpec((HIDDEN, L * DIM), lambda t: (0, 0)),
            pl.BlockSpec((HIDDEN, 1), lambda t: (0, 0)),
            pl.BlockSpec((HIDDEN, HIDDEN), lambda t: (0, 0)),
            pl.BlockSpec((HIDDEN, 1), lambda t: (0, 0)),
            pl.BlockSpec((8, HIDDEN), lambda t: (0, 0)),
            pl.BlockSpec((8, 1), lambda t: (0, 0)),
        ],
        out_specs=pl.BlockSpec((8, SD), lambda t: (0, t)),
        out_shape=jax.ShapeDtypeStruct((8, TPAD), jnp.float32),
    )(raw, w_t, W1t, b1.reshape(HIDDEN, 1), W2t, b2.reshape(HIDDEN, 1),
      W3t, b3p.reshape(8, 1))

    return out[0, :T].reshape(T, 1)


# speedup vs baseline: 1.8133x; 1.0236x over previous
import functools

import jax
import jax.numpy as jnp
from jax import lax
from jax.experimental import pallas as pl
from jax.experimental.pallas import tpu as pltpu
from jax.experimental.pallas import tpu_sc as plsc

L = 10
DIM = 4
LOG2T = 19
TSIZE = 2 ** LOG2T
BASE_RES = 16.0
SCALE_MULTI = 0.5
HIDDEN = 64

NC = 2
NS = 16
NW = NC * NS

TR = 3968
TPAD = TR * 128


def _sc_gather_planes(tab1d, idx2d, chunk, nplanes, stride):
    n = idx2d.shape[0] * 128
    per_w = n // NW
    assert per_w % chunk == 0 and chunk % 128 == 0
    nchunks = per_w // chunk
    nrow = chunk // 128
    mesh = plsc.VectorSubcoreMesh(core_axis_name="c", subcore_axis_name="s")

    @functools.partial(
        pl.kernel,
        out_type=jax.ShapeDtypeStruct((nplanes * n,), jnp.float32),
        mesh=mesh,
        scratch_types=[
            pltpu.VMEM((2, nrow, 128), jnp.int32),
            pltpu.VMEM((2, nplanes, chunk), jnp.int32),
            pltpu.VMEM((2, nplanes, chunk), jnp.float32),
            pltpu.SemaphoreType.DMA,
        ],
        compiler_params=pltpu.CompilerParams(use_tc_tiling_on_sc=False,
                                             needs_layout_passes=False),
    )
    def k(tab_hbm, idx_hbm, out_hbm, idx_v, gidx, planes_v, sem):
        wid = lax.axis_index("s") * NC + lax.axis_index("c")
        base = wid * per_w

        def stage(j, b):
            off = base + j * chunk
            pltpu.sync_copy(idx_hbm.at[pl.ds(off // 128, nrow)], idx_v.at[b])

            def expand(g, c2):
                for v in range(8):
                    h = idx_v[b, g, pl.ds(v * 16, 16)]
                    for d in range(nplanes):
                        gidx[b, d, pl.ds(g * 128 + v * 16, 16)] = h + d * stride
                return c2

            lax.fori_loop(0, nrow, expand, 0)
            for d in range(nplanes):
                pltpu.async_copy(tab_hbm.at[gidx.at[b, d]],
                                 planes_v.at[b, d], sem)

        stage(0, 0)

        def body(j, carry):
            b = lax.rem(j, 2)
            nb = 1 - b

            @pl.when(j + 1 < nchunks)
            def _():
                stage(j + 1, nb)

            off = base + j * chunk
            for d in range(nplanes):
                pltpu.make_async_copy(tab_hbm.at[gidx.at[b, d]],
                                      planes_v.at[b, d], sem).wait()
            for d in range(nplanes):
                pltpu.sync_copy(planes_v.at[b, d],
                                out_hbm.at[pl.ds(d * n + off, chunk)])
            return carry

        lax.fori_loop(0, nchunks, body, 0)

    return k(tab1d, idx2d)


def _erf(x):
    t = 1.0 / (1.0 + 0.3275911 * x)
    poly = t * (0.254829592 + t * (-0.284496736 + t * (1.421413741
                + t * (-1.453152027 + t * 1.061405429))))
    return 1.0 - poly * jnp.exp(-x * x)


def _geom_kernel(xyz_ref, cs_ref, idx_ref, w_ref):
    inv_s = 1.0 / cs_ref[3]
    cen = (cs_ref[0], cs_ref[1], cs_ref[2])
    a = [(xyz_ref[0, i] - cen[i]) * inv_s for i in range(3)]
    d1 = [(xyz_ref[1, i] - cen[i]) * inv_s - a[i] for i in range(3)]
    d2 = [(xyz_ref[2, i] - cen[i]) * inv_s - a[i] for i in range(3)]
    d3 = [(xyz_ref[3, i] - cen[i]) * inv_s - a[i] for i in range(3)]
    eps = jnp.float32(1e-8)
    a11 = 2.0 * d1[0] + eps
    a12 = 2.0 * d1[1]
    a13 = 2.0 * d1[2]
    a21 = 2.0 * d2[0]
    a22 = 2.0 * d2[1] + eps
    a23 = 2.0 * d2[2]
    a31 = 2.0 * d3[0]
    a32 = 2.0 * d3[1]
    a33 = 2.0 * d3[2] + eps
    r1 = d1[0] * d1[0] + d1[1] * d1[1] + d1[2] * d1[2]
    r2 = d2[0] * d2[0] + d2[1] * d2[1] + d2[2] * d2[2]
    r3 = d3[0] * d3[0] + d3[1] * d3[1] + d3[2] * d3[2]
    c11 = a22 * a33 - a23 * a32
    c12 = a23 * a31 - a21 * a33
    c13 = a21 * a32 - a22 * a31
    det = a11 * c11 + a12 * c12 + a13 * c13
    inv = 1.0 / det
    ox = (r1 * c11 + r2 * (a13 * a32 - a12 * a33) + r3 * (a12 * a23 - a13 * a22)) * inv
    oy = (r1 * c12 + r2 * (a11 * a33 - a13 * a31) + r3 * (a13 * a21 - a11 * a23)) * inv
    oz = (r1 * c13 + r2 * (a12 * a31 - a11 * a32) + r3 * (a11 * a22 - a12 * a21)) * inv
    cx = a[0] + ox
    cy = a[1] + oy
    cz = a[2] + oz
    radius = jnp.sqrt(ox * ox + oy * oy + oz * oz)
    nrm = jnp.sqrt(cx * cx + cy * cy + cz * cz)
    safe = jnp.maximum(nrm, 1.0)
    fac = jnp.where(nrm <= 1.0, 1.0, (2.0 - 1.0 / safe) / safe)
    cr = radius / (safe * safe) * SCALE_MULTI
    xs0 = jnp.clip((cx * fac * 0.5 + 1.0) * 0.5, 0.0, 1.0 - 1e-6)
    xs1 = jnp.clip((cy * fac * 0.5 + 1.0) * 0.5, 0.0, 1.0 - 1e-6)
    xs2 = jnp.clip((cz * fac * 0.5 + 1.0) * 0.5, 0.0, 1.0 - 1e-6)
    p1 = jnp.int32(-1640531535)
    p2 = jnp.int32(805459861)
    mask = jnp.int32(TSIZE - 1)
    for l in range(L):
        res = BASE_RES * (2.0 ** l)
        gx = xs0 * res
        gy = xs1 * res
        gz = xs2 * res
        fx = jnp.floor(gx)
        fy = jnp.floor(gy)
        fz = jnp.floor(gz)
        wx1 = gx - fx
        wy1 = gy - fy
        wz1 = gz - fz
        wx0 = 1.0 - wx1
        wy0 = 1.0 - wy1
        wz0 = 1.0 - wz1
        ix = fx.astype(jnp.int32)
        iy = fy.astype(jnp.int32)
        iz = fz.astype(jnp.int32)
        hx = (ix, ix + 1)
        hy0 = iy * p1
        hy = (hy0, hy0 + p1)
        hz0 = iz * p2
        hz = (hz0, hz0 + p2)
        wxs = (wx0, wx1)
        wys = (wy0, wy1)
        wzs = (wz0, wz1)
        m = jnp.maximum(jnp.float32(8.0 * l) * cr, 1e-12)
        scal = _erf(jax.lax.rsqrt(m))
        base = jnp.int32(l * TSIZE)
        for c in range(8):
            bx = (c >> 0) & 1
            by = (c >> 1) & 1
            bz = (c >> 2) & 1
            h = ((hx[bx] ^ hy[by] ^ hz[bz]) & mask) + base
            wc = wxs[bx] * wys[by] * wzs[bz] * scal
            idx_ref[c * L + l] = h
            w_ref[c * L + l] = wc


def _mlp_kernel(raw_ref, w_ref,
                W1t_ref, b1_ref, W2t_ref, b2_ref, W3t_ref, b3_ref, out_ref):
    s = w_ref.shape[1]
    j_i = lax.broadcasted_iota(jnp.int32, (4 * L, 8 * L), 0)
    cl_i = lax.broadcasted_iota(jnp.int32, (4 * L, 8 * L), 1)
    w = w_ref[:, :]
    featsT = jnp.zeros((4 * L, s), jnp.float32)
    for d in range(4):
        Gt = ((j_i % 4 == d) & (j_i // 4 == cl_i % L)).astype(jnp.float32)
        featsT = featsT + jnp.dot(Gt, w * raw_ref[d],
                                  preferred_element_type=jnp.float32)
    scale = jnp.float32(1.0507009873554805)
    alpha = jnp.float32(1.6732632423543772)

    def selu(x):
        return scale * jnp.where(x > 0, x, alpha * (jnp.exp(x) - 1.0))

    h = selu(jnp.dot(W1t_ref[:, :], featsT, preferred_element_type=jnp.float32)
             + b1_ref[:, :])
    h = selu(jnp.dot(W2t_ref[:, :], h, preferred_element_type=jnp.float32)
             + b2_ref[:, :])
    out_ref[:, :] = (jnp.dot(W3t_ref[:, :], h, preferred_element_type=jnp.float32)
                     + b3_ref[:, :])


def kernel(vertices, indices, tables, W1, b1, W2, b2, W3, b3, center, scene_scaling):
    T = indices.shape[0]
    idx32 = indices.astype(jnp.int32)
    idx_pad = jnp.pad(idx32, ((0, TPAD - T), (0, 0)))
    VP = 200064
    verts1d = jnp.pad(vertices, ((0, VP - vertices.shape[0]), (0, 0)))
    verts1d = verts1d.transpose(1, 0).reshape(3, VP // 128, 128).reshape(-1)

    nv = 4 * TPAD
    vflat = idx_pad.reshape(nv // 128, 128)
    vgp = _sc_gather_planes(verts1d, vflat, chunk=2048, nplanes=3, stride=VP)
    xyz = vgp.reshape(3, TPAD, 4).transpose(2, 0, 1)
    xyz = xyz.reshape(4, 3, TR, 128)
    cs = jnp.concatenate([center.reshape(3), scene_scaling.reshape(1)])

    S = 64
    grid = TR // S
    idx80, w80 = pl.pallas_call(
        _geom_kernel,
        grid=(grid,),
        in_specs=[
            pl.BlockSpec((4, 3, S, 128), lambda t: (0, 0, t, 0)),
            pl.BlockSpec(memory_space=pltpu.SMEM),
        ],
        out_specs=[
            pl.BlockSpec((8 * L, S, 128), lambda t: (0, t, 0)),
            pl.BlockSpec((8 * L, S, 128), lambda t: (0, t, 0)),
        ],
        out_shape=[
            jax.ShapeDtypeStruct((8 * L, TR, 128), jnp.int32),
            jax.ShapeDtypeStruct((8 * L, TR, 128), jnp.float32),
        ],
    )(xyz, cs)

    nt = TPAD * 8 * L
    idx_t = idx80.reshape(nt // 128, 128)
    w_t = w80.reshape(8 * L, TPAD)

    LT = L * TSIZE
    tab1d = tables.reshape(LT, 4).transpose(1, 0)
    tab1d = tab1d.reshape(4, LT // 128, 128).reshape(-1)
    rawp = _sc_gather_planes(tab1d, idx_t, chunk=5120, nplanes=4, stride=LT)
    raw = rawp.reshape(4, 8 * L, TPAD)

    SD = 512
    W1t = W1.transpose(1, 0)
    W2t = W2.transpose(1, 0)
    W3t = jnp.pad(W3, ((0, 0), (0, 7))).transpose(1, 0)
    b3p = jnp.pad(b3, (0, 7))
    out = pl.pallas_call(
        _mlp_kernel,
        grid=(TPAD // SD,),
        in_specs=[
            pl.BlockSpec((4, 8 * L, SD), lambda t: (0, 0, t)),
            pl.BlockSpec((8 * L, SD), lambda t: (0, t)),
            pl.BlockSpec((HIDDEN, L * DIM), lambda t: (0, 0)),
            pl.BlockSpec((HIDDEN, 1), lambda t: (0, 0)),
            pl.BlockSpec((HIDDEN, HIDDEN), lambda t: (0, 0)),
            pl.BlockSpec((HIDDEN, 1), lambda t: (0, 0)),
            pl.BlockSpec((8, HIDDEN), lambda t: (0, 0)),
            pl.BlockSpec((8, 1), lambda t: (0, 0)),
        ],
        out_specs=pl.BlockSpec((8, SD), lambda t: (0, t)),
        out_shape=jax.ShapeDtypeStruct((8, TPAD), jnp.float32),
    )(raw, w_t, W1t, b1.reshape(HIDDEN, 1), W2t, b2.reshape(HIDDEN, 1),
      W3t, b3p.reshape(8, 1))

    return out[0, :T].reshape(T, 1)
